# baseline (speedup 1.0000x reference)
import functools

import numpy as _np

import jax
import jax.numpy as jnp
from jax import lax
from jax.experimental import pallas as pl
from jax.experimental.pallas import tpu as pltpu
from jax.experimental.pallas import tpu_sc as plsc

N = 10000
E = 320000
D = 128
C = 11
CP = 16
W = 32
GT = 80
GROUP = 128
EP = W * GT * GROUP
ROWS_PER_TILE = N // 16
INV_SQRT_C = 1.0 / (C ** 0.5)
_NBUF = 4


_R = 8
_NB = N // _R
_BB = _NB

_SEL_NP = _np.zeros((_R * CP, _R * CP), _np.float32)
for _j in range(_R * CP):
    _SEL_NP[(_j // CP) * CP + C, _j] = 1.0


def _project_body(x_ref, w_ref, b_ref, q_ref, k_ref, v_ref, s_ref):
    z = jnp.dot(x_ref[...], w_ref[...], preferred_element_type=jnp.float32)
    z = z + b_ref[...]
    q_ref[...] = z[:, 0:128]
    k_ref[...] = z[:, 128:256]
    v_ref[...] = z[:, 256:384]
    s_ref[...] = z[:, 384:512]


def _tc_project(xb, wblk, bblk, in_dim):
    out = jax.ShapeDtypeStruct((_NB, _R * CP), jnp.float32)
    return pl.pallas_call(
        _project_body,
        grid=(_NB // _BB,),
        in_specs=[
            pl.BlockSpec((_BB, in_dim), lambda i: (i, 0)),
            pl.BlockSpec((in_dim, 4 * _R * CP), lambda i: (0, 0)),
            pl.BlockSpec((1, 4 * _R * CP), lambda i: (0, 0)),
        ],
        out_specs=[pl.BlockSpec((_BB, _R * CP), lambda i: (i, 0))] * 4,
        out_shape=[out, out, out, out],
    )(xb, wblk, bblk)


def _merge_normalize(acc_ref, mx_ref, skip_ref, sel_ref):
    g0 = jnp.max(mx_ref[0])
    g1 = jnp.max(mx_ref[1])
    g = jnp.maximum(g0, g1)
    a = acc_ref[0] * jnp.exp(g0 - g) + acc_ref[1] * jnp.exp(g1 - g)
    wb = jnp.dot(a, sel_ref[...], preferred_element_type=jnp.float32)
    h = a / (wb + 1e-16) + skip_ref[...]
    h = jnp.maximum(h, 0.0)
    col = lax.broadcasted_iota(jnp.int32, h.shape, 1)
    return jnp.where(col % CP < C, h, 0.0)


def _mid_body(acc_ref, mx_ref, skip_ref, sel_ref, w_ref, b_ref,
              q_ref, k_ref, v_ref, s_ref):
    h = _merge_normalize(acc_ref, mx_ref, skip_ref, sel_ref)
    z = jnp.dot(h, w_ref[...], preferred_element_type=jnp.float32)
    z = z + b_ref[...]
    q_ref[...] = z[:, 0:128]
    k_ref[...] = z[:, 128:256]
    v_ref[...] = z[:, 256:384]
    s_ref[...] = z[:, 384:512]


def _tc_mid(acc, mx, skip, sel, wblk, bblk):
    out = jax.ShapeDtypeStruct((_NB, _R * CP), jnp.float32)
    return pl.pallas_call(
        _mid_body,
        grid=(_NB // _BB,),
        in_specs=[
            pl.BlockSpec((2, _BB, _R * CP), lambda i: (0, i, 0)),
            pl.BlockSpec((2, 16), lambda i: (0, 0)),
            pl.BlockSpec((_BB, _R * CP), lambda i: (i, 0)),
            pl.BlockSpec((_R * CP, _R * CP), lambda i: (0, 0)),
            pl.BlockSpec((_R * CP, 4 * _R * CP), lambda i: (0, 0)),
            pl.BlockSpec((1, 4 * _R * CP), lambda i: (0, 0)),
        ],
        out_specs=[pl.BlockSpec((_BB, _R * CP), lambda i: (i, 0))] * 4,
        out_shape=[out, out, out, out],
    )(acc, mx, skip, sel, wblk, bblk)


def _final_body(acc_ref, mx_ref, skip_ref, sel_ref, wo_ref, bo_ref, y_ref):
    h = _merge_normalize(acc_ref, mx_ref, skip_ref, sel_ref)
    z = jnp.dot(h, wo_ref[...], preferred_element_type=jnp.float32)
    z = z + bo_ref[...]
    y_ref[...] = 1.0 / (1.0 + jnp.exp(-z))


def _tc_final(acc, mx, skip, sel, wo_blk, bo_blk):
    return pl.pallas_call(
        _final_body,
        grid=(_NB // _BB,),
        in_specs=[
            pl.BlockSpec((2, _BB, _R * CP), lambda i: (0, i, 0)),
            pl.BlockSpec((2, 16), lambda i: (0, 0)),
            pl.BlockSpec((_BB, _R * CP), lambda i: (i, 0)),
            pl.BlockSpec((_R * CP, _R * CP), lambda i: (0, 0)),
            pl.BlockSpec((_R * CP, _R * D), lambda i: (0, 0)),
            pl.BlockSpec((1, _R * D), lambda i: (0, 0)),
        ],
        out_specs=pl.BlockSpec((_BB, _R * D), lambda i: (i, 0)),
        out_shape=jax.ShapeDtypeStruct((_NB, _R * D), jnp.float32),
    )(acc, mx, skip, sel, wo_blk, bo_blk)


@functools.lru_cache(maxsize=1)
def _sc_mesh():
    return plsc.VectorSubcoreMesh(
        core_axis_name="c", subcore_axis_name="s", num_cores=2, num_subcores=16)


def _sc_layer_body(q_hbm, k_hbm, v_hbm, ei_hbm,
                   acc_out, mx_out,
                   src_v, dst_v, alpha_vt, qr_all, kr_all, vr_all, msg_all,
                   mxv, mx_v, tmp,
                   acc_spmem, mx_spmem, *sems):
    qrs = [qr_all.at[b] for b in range(_NBUF)]
    krs = [kr_all.at[b] for b in range(_NBUF)]
    vrs = [vr_all.at[b] for b in range(_NBUF)]
    msgs = [msg_all.at[b] for b in range(_NBUF)]
    sems_q = sems[0:_NBUF]
    sems_k = sems[_NBUF:2 * _NBUF]
    sems_v = sems[2 * _NBUF:3 * _NBUF]
    sems_s = sems[3 * _NBUF:4 * _NBUF]

    c = lax.axis_index("c")
    s = lax.axis_index("s")
    wid = c * 16 + s
    base_g = wid * GT
    lane = lax.iota(jnp.int32, 16)
    zero16 = jnp.zeros((16,), jnp.float32)

    pltpu.sync_copy(ei_hbm.at[0, pl.ds(base_g, GT)], src_v)
    pltpu.sync_copy(ei_hbm.at[1, pl.ds(base_g, GT)], dst_v)

    def start_qk(g, b):
        pltpu.make_async_copy(q_hbm.at[dst_v.at[g]], qrs[b], sems_q[b]).start()
        pltpu.make_async_copy(k_hbm.at[src_v.at[g]], krs[b], sems_k[b]).start()

    def wait_qk(g, b):
        pltpu.make_async_copy(q_hbm.at[dst_v.at[g]], qrs[b], sems_q[b]).wait()
        pltpu.make_async_copy(k_hbm.at[src_v.at[g]], krs[b], sems_k[b]).wait()

    def compute_alpha(g, b, mx):
        qr = qrs[b]
        kr = krs[b]
        for sub in range(8):
            idx = lane + (sub * 16)
            acc = jnp.zeros((16,), jnp.float32)
            for ch in range(C):
                chv = jnp.full((16,), ch, jnp.int32)
                qc = plsc.load_gather(qr, [idx, chv])
                kc = plsc.load_gather(kr, [idx, chv])
                acc = acc + qc * kc
            acc = acc * INV_SQRT_C
            alpha_vt[g, pl.ds(sub * 16, 16)] = acc
            mx = jnp.maximum(mx, acc)
        return mx

    for b in range(_NBUF - 1):
        start_qk(b, b)

    def zrow(i, _):
        tmp[i, :] = zero16
        return 0
    lax.fori_loop(0, ROWS_PER_TILE, zrow, 0)

    def zmsg(i, _):
        for b in range(_NBUF):
            msgs[b][i, :] = zero16
        return 0
    lax.fori_loop(0, GROUP, zmsg, 0)

    pltpu.sync_copy(tmp, acc_spmem.at[pl.ds(s * ROWS_PER_TILE, ROWS_PER_TILE)])

    def quad1(it, mx):
        for j in range(_NBUF):
            g = _NBUF * it + j

            @pl.when(g + _NBUF - 1 < GT)
            def _():
                start_qk(g + _NBUF - 1, (j + _NBUF - 1) % _NBUF)

            wait_qk(g, j)
            mx = compute_alpha(g, j, mx)
        return mx

    mx = lax.fori_loop(0, GT // _NBUF, quad1,
                       jnp.full((16,), -1e30, jnp.float32))

    mx_v[...] = mx
    pltpu.sync_copy(mx_v, mx_spmem.at[s])

    def start_v(g, b):
        pltpu.make_async_copy(v_hbm.at[src_v.at[g]], vrs[b], sems_v[b]).start()

    for b in range(_NBUF - 1):
        start_v(b, b)

    plsc.subcore_barrier()

    pltpu.sync_copy(mx_spmem, mxv)
    m = jnp.full((16,), -1e30, jnp.float32)
    for i in range(16):
        m = jnp.maximum(m, mxv[i, :])
    gmax = jnp.max(m)

    @pl.when(s == 0)
    def _():
        mx_v[...] = m
        pltpu.sync_copy(mx_v, mx_out.at[c])

    def wait_v(g, b):
        pltpu.make_async_copy(v_hbm.at[src_v.at[g]], vrs[b], sems_v[b]).wait()

    def wait_scatter(b):
        pltpu.make_async_copy(msgs[b], acc_spmem.at[dst_v.at[0]],
                              sems_s[b]).wait()

    def compute_msg(g, b):
        vr = vrs[b]
        msg = msgs[b]
        for sub in range(8):
            idx = lane + (sub * 16)
            a = alpha_vt[g, pl.ds(sub * 16, 16)]
            ae = jnp.exp(a - gmax)
            gid = (base_g + g) * GROUP + sub * 16 + lane
            ae = jnp.where(gid < E, ae, 0.0)
            for ch in range(C):
                chv = jnp.full((16,), ch, jnp.int32)
                vc = plsc.load_gather(vr, [idx, chv])
                plsc.store_scatter(msg, [idx, chv], vc * ae)
            plsc.store_scatter(msg, [idx, jnp.full((16,), C, jnp.int32)], ae)
        pltpu.async_copy(msg, acc_spmem.at[dst_v.at[g]], sems_s[b], add=True)

    def quad2(it, _):
        for j in range(_NBUF):
            g = _NBUF * it + j

            @pl.when(g + _NBUF - 1 < GT)
            def _():
                start_v(g + _NBUF - 1, (j + _NBUF - 1) % _NBUF)

            wait_v(g, j)

            @pl.when(it > 0)
            def _():
                wait_scatter(j)

            compute_msg(g, j)
        return 0

    lax.fori_loop(0, GT // _NBUF, quad2, 0)
    for b in range(_NBUF):
        wait_scatter(b)
    plsc.subcore_barrier()

    pltpu.sync_copy(acc_spmem.at[pl.ds(s * ROWS_PER_TILE, ROWS_PER_TILE)], tmp)
    pltpu.sync_copy(tmp, acc_out.at[c, pl.ds(s * ROWS_PER_TILE, ROWS_PER_TILE)])


@functools.lru_cache(maxsize=1)
def _sc_layer():
    return pl.kernel(
        _sc_layer_body,
        out_type=[
            jax.ShapeDtypeStruct((2, N, CP), jnp.float32),
            jax.ShapeDtypeStruct((2, 16), jnp.float32),
        ],
        mesh=_sc_mesh(),
        scratch_types=[
            pltpu.VMEM((GT, GROUP), jnp.int32),
            pltpu.VMEM((GT, GROUP), jnp.int32),
            pltpu.VMEM((GT, GROUP), jnp.float32),
            pltpu.VMEM((_NBUF, GROUP, CP), jnp.float32),
            pltpu.VMEM((_NBUF, GROUP, CP), jnp.float32),
            pltpu.VMEM((_NBUF, GROUP, CP), jnp.float32),
            pltpu.VMEM((_NBUF, GROUP, CP), jnp.float32),
            pltpu.VMEM((16, 16), jnp.float32),
            pltpu.VMEM((16,), jnp.float32),
            pltpu.VMEM((ROWS_PER_TILE, CP), jnp.float32),
            pltpu.VMEM_SHARED((N, CP), jnp.float32),
            pltpu.VMEM_SHARED((16, 16), jnp.float32),
        ] + [pltpu.SemaphoreType.DMA] * (4 * _NBUF),
        compiler_params=pltpu.CompilerParams(
            use_tc_tiling_on_sc=False, needs_layout_passes=False),
    )


_EYE8 = _np.eye(_R, dtype=_np.float32)


def _blk_weight(w, in_rows):
    wp = jnp.pad(w, ((0, in_rows - w.shape[0]), (0, CP - w.shape[1])))
    return jnp.kron(_EYE8, wp)


def _pack_weights(wq, bq, wk, bk, wv, bv, ws, bs, in_rows):
    wblk = jnp.concatenate(
        [_blk_weight(w, in_rows) for w in (wq, wk, wv, ws)], axis=1)
    bblk = jnp.concatenate(
        [jnp.tile(jnp.pad(b, (0, CP - C)), _R) for b in (bq, bk, bv, bs)]
    ).reshape(1, 4 * _R * CP)
    return wblk, bblk


def kernel(x, edge_index, Wq1, bq1, Wk1, bk1, Wv1, bv1, Ws1, bs1,
           Wq2, bq2, Wk2, bk2, Wv2, bv2, Ws2, bs2, Wo, bo):
    ei = jnp.pad(edge_index, ((0, 0), (0, EP - E))).reshape(2, W * GT, GROUP)

    wblk1, bblk1 = _pack_weights(Wq1, bq1, Wk1, bk1, Wv1, bv1, Ws1, bs1, D)
    wblk2, bblk2 = _pack_weights(Wq2, bq2, Wk2, bk2, Wv2, bv2, Ws2, bs2, CP)
    wo_blk = jnp.kron(_EYE8, jnp.pad(Wo, ((0, CP - C), (0, 0))))
    bo_blk = jnp.tile(bo, _R).reshape(1, _R * D)
    sel = jnp.asarray(_SEL_NP)

    sc_layer = _sc_layer()

    def t16(a):
        return a.reshape(N, CP)

    xb = x.reshape(_NB, _R * D)
    q1, k1, v1, s1 = _tc_project(xb, wblk1, bblk1, _R * D)
    acc1, mx1 = sc_layer(t16(q1), t16(k1), t16(v1), ei)

    q2, k2, v2, s2 = _tc_mid(acc1.reshape(2, _NB, _R * CP), mx1,
                             s1, sel, wblk2, bblk2)
    acc2, mx2 = sc_layer(t16(q2), t16(k2), t16(v2), ei)

    y = _tc_final(acc2.reshape(2, _NB, _R * CP), mx2, s2, sel, wo_blk, bo_blk)
    return y.reshape(N, D)
